# baseline (speedup 1.0000x reference)
import functools

import jax
import jax.numpy as jnp
from jax import lax
from jax.experimental import pallas as pl
from jax.experimental.pallas import tpu as pltpu
from jax.experimental.pallas import tpu_sc as plsc

_H = 128
_NCE, _NPE, _NSE = 100000, 50000, 1000
_NCP, _NPP, _NSP = 100096, 50176, 1024
_EUNIT = 32768
_BIG = 1 << 30
_K = 4


def _ceil_to(x, m):
    return -(-x // m) * m


@functools.cache
def _make_segsum(ns_pad, ds, f, ep, racc):
    cr = ep // 4096
    ng = cr // _K
    rps = racc // 16
    zr = max(16, 4096 // ds)
    nzb, nzs = rps // zr, (rps % zr) // 16
    mesh = plsc.VectorSubcoreMesh(core_axis_name="c", subcore_axis_name="s")

    scratch = [
        pltpu.VMEM((2, 2 * _K, 128), jnp.int32),
        pltpu.VMEM((2, _K, 128, ds), jnp.float32),
        pltpu.VMEM((zr, ds), jnp.float32),
        pltpu.VMEM_SHARED((racc + 16, ds), jnp.float32),
        pltpu.SemaphoreType.DMA,
        pltpu.SemaphoreType.DMA,
    ]
    if f > 1:
        scratch.append(pltpu.VMEM((2, _K, 128), jnp.int32))

    @functools.partial(
        pl.kernel, mesh=mesh,
        out_type=jax.ShapeDtypeStruct((2 * f * racc, ds), jnp.float32),
        scratch_types=scratch,
        compiler_params=pltpu.CompilerParams(use_tc_tiling_on_sc=False),
    )
    def seg(table, epk, out, ebuf, rows, zt, acc, sem_g, sem_s, *maybe_srcf):
        cid = lax.axis_index("c")
        sid = lax.axis_index("s")
        base_row = cid * (ep // 256) + sid * cr

        for r in range(zr):
            for j in range(ds // 16):
                zt[r, pl.ds(j * 16, 16)] = jnp.zeros((16,), jnp.float32)

        def zero_slice():
            def zb(k, _):
                pltpu.sync_copy(zt, acc.at[pl.ds(sid * rps + k * zr, zr)])
                return 0
            lax.fori_loop(0, nzb, zb, 0)
            for k in range(nzs):
                pltpu.sync_copy(
                    zt.at[pl.ds(0, 16)],
                    acc.at[pl.ds(sid * rps + nzb * zr + k * 16, 16)])

        def stage_edges(par, g):
            pltpu.sync_copy(epk.at[pl.ds(2 * base_row + g * 8, 8)],
                            ebuf.at[par])

        def calc_srcf(par, foff):
            srcf = maybe_srcf[0]
            for j in range(_K):
                for v in range(8):
                    sl = pl.ds(v * 16, 16)
                    srcf[par, j, sl] = ebuf[par, j, sl] + foff

        def fire_gathers(par):
            if f > 1:
                idx = maybe_srcf[0]
                for j in range(_K):
                    pltpu.async_copy(table.at[idx.at[par, j]],
                                     rows.at[par, j], sem_g)
            else:
                for j in range(_K):
                    pltpu.async_copy(table.at[ebuf.at[par, j]],
                                     rows.at[par, j], sem_g)

        def drain_gathers(par):
            for j in range(_K):
                pltpu.make_async_copy(table.at[pl.ds(0, 128)],
                                      rows.at[par, j], sem_g).wait()

        def fire_scatters(par):
            for j in range(_K):
                pltpu.async_copy(rows.at[par, j],
                                 acc.at[ebuf.at[par, _K + j]], sem_s,
                                 add=True)

        def drain_scatters(par):
            for j in range(_K):
                pltpu.make_async_copy(rows.at[par, j],
                                      acc.at[pl.ds(0, 128)], sem_s).wait()

        zero_slice()
        plsc.subcore_barrier()

        def pass_body(p, _):
            foff = p * ns_pad
            stage_edges(0, 0)
            if f > 1:
                calc_srcf(0, foff)
            fire_gathers(0)

            def pair_body(i, _):
                @pl.when(i > 0)
                def _():
                    drain_scatters(1)
                drain_gathers(0)
                stage_edges(1, 2 * i + 1)
                if f > 1:
                    calc_srcf(1, foff)
                fire_gathers(1)
                fire_scatters(0)
                drain_scatters(0)
                @pl.when(i < ng // 2 - 1)
                def _():
                    drain_gathers(1)
                    stage_edges(0, 2 * i + 2)
                    if f > 1:
                        calc_srcf(0, foff)
                    fire_gathers(0)
                    fire_scatters(1)
                return 0

            lax.fori_loop(0, ng // 2, pair_body, 0)
            drain_gathers(1)
            fire_scatters(1)
            drain_scatters(1)
            plsc.subcore_barrier()

            pltpu.sync_copy(
                acc.at[pl.ds(sid * rps, rps)],
                out.at[pl.ds((cid * f + p) * racc + sid * rps, rps)])
            zero_slice()
            plsc.subcore_barrier()
            return 0

        lax.fori_loop(0, f, pass_body, 0)

    return seg


_BLK = 256


def _full(shape):
    return pl.BlockSpec(shape, lambda i: (0, 0))


def _rows(w):
    return pl.BlockSpec((_BLK, w), lambda i: (i, 0))


@functools.cache
def _make_pack16(n, d, npad):
    def body(x, o):
        xb = x[...]
        one = jnp.ones((_BLK, 1), jnp.float32)
        zero = jnp.zeros((_BLK, 15 - d), jnp.float32)
        o[...] = jnp.concatenate([xb, one, zero], axis=1)

    return pl.pallas_call(
        body,
        grid=(npad // _BLK,),
        in_specs=[pl.BlockSpec((_BLK, d), lambda i: (i, 0))],
        out_specs=_rows(16),
        out_shape=jax.ShapeDtypeStruct((npad, 16), jnp.float32),
    )


@functools.cache
def _make_epk(n, ep, racc):
    def body(eb, o):
        g = pl.program_id(0)
        col = lax.broadcasted_iota(jnp.int32, (1, 512), 1) + g * 512
        m = col < n
        s = jnp.where(m, eb[0:1, :], 0)
        d = jnp.where(m, jnp.minimum(eb[1:2, :], racc), racc)
        o[...] = jnp.concatenate(
            [s.reshape(4, 128), d.reshape(4, 128)], axis=0)

    return pl.pallas_call(
        body,
        grid=(ep // 512,),
        in_specs=[pl.BlockSpec((2, 512), lambda g: (0, g))],
        out_specs=pl.BlockSpec((8, 128), lambda g: (g, 0)),
        out_shape=jax.ShapeDtypeStruct((ep // 64, 128), jnp.int32),
    )


def _layer_tail(h, g, b):
    mu = jnp.mean(h, axis=1, keepdims=True)
    var = jnp.mean((h - mu) ** 2, axis=1, keepdims=True)
    hn = (h - mu) * lax.rsqrt(var + 1e-5) * g + b
    return jnp.maximum(hn, 0.0)


@functools.cache
def _make_stage_a(npad, dc1, dc2):
    def body(x16, a1c0, a1c1, a2c0, a2c1, aw1, aw2, bwh, cv, cst, g, b, o):
        a1 = a1c0[...] + a1c1[...]
        a2 = a2c0[...] + a2c1[...]
        c1 = a1[:, dc1:dc1 + 1]
        c2 = a2[:, dc2:dc2 + 1]
        na1 = a1 / jnp.maximum(c1, 1.0)
        na2 = a2 / jnp.maximum(c2, 1.0)
        m1 = (c1 > 0).astype(jnp.float32)
        m2 = (c2 > 0).astype(jnp.float32)
        h = (jnp.dot(na1, aw1[...], preferred_element_type=jnp.float32)
             + jnp.dot(na2, aw2[...], preferred_element_type=jnp.float32)
             + jnp.dot(x16[...], bwh[...], preferred_element_type=jnp.float32)
             + m1 * cv[0:1, :] + m2 * cv[1:2, :] + cst[...])
        o[...] = _layer_tail(h, g[...], b[...])

    return pl.pallas_call(
        body,
        grid=(npad // _BLK,),
        in_specs=[_rows(16)] * 5 + [_full((16, _H))] * 3
        + [_full((2, _H)), _full((1, _H)), _full((1, _H)), _full((1, _H))],
        out_specs=_rows(_H),
        out_shape=jax.ShapeDtypeStruct((npad, _H), jnp.float32),
    )


@functools.cache
def _make_stage_b(npad, dc1, dc2, ds, f):
    def body(*refs):
        x = refs[0]
        sl = refs[1:1 + 4 * f]
        k1c0, k1c1, k2c0, k2c1 = refs[1 + 4 * f:5 + 4 * f]
        wl1, wl2, wrh, cst, g, b = refs[5 + 4 * f:11 + 4 * f]
        o = refs[11 + 4 * f]

        def wide(grp):
            parts = sl[grp * f:(grp + 1) * f]
            if f == 1:
                return parts[0][...]
            return jnp.concatenate([p[...] for p in parts], axis=1)

        c1 = (k1c0[...] + k1c1[...])[:, dc1:dc1 + 1]
        c2 = (k2c0[...] + k2c1[...])[:, dc2:dc2 + 1]
        na1 = (wide(0) + wide(1)) / jnp.maximum(c1, 1.0)
        na2 = (wide(2) + wide(3)) / jnp.maximum(c2, 1.0)
        h = (jnp.dot(na1, wl1[...], preferred_element_type=jnp.float32)
             + jnp.dot(na2, wl2[...], preferred_element_type=jnp.float32)
             + jnp.dot(x[...], wrh[...], preferred_element_type=jnp.float32)
             + cst[...])
        o[...] = _layer_tail(h, g[...], b[...])

    return pl.pallas_call(
        body,
        grid=(npad // _BLK,),
        in_specs=[_rows(_H)] + [_rows(ds)] * (4 * f) + [_rows(16)] * 4
        + [_full((_H, _H))] * 3
        + [_full((1, _H)), _full((1, _H)), _full((1, _H))],
        out_specs=_rows(_H),
        out_shape=jax.ShapeDtypeStruct((npad, _H), jnp.float32),
    )


_SRC_T = (0, 1, 0, 2, 1, 2)
_DST_T = (1, 0, 2, 0, 2, 1)
_DCOL = (6, 4, 3)
_INC = ((1, 3), (0, 5), (2, 4))
_L2DS = ((16, 8), (16, 8), (64, 2))


def _feat_split(x, ds, f):
    if f == 1:
        return x
    n = x.shape[0]
    return x.reshape(n, f, ds).transpose(1, 0, 2).reshape(f * n, ds)


def _unsplit(out, racc, ds, f):
    o = out.reshape(2, f, racc, ds)
    if f == 1:
        return o[0, 0], o[1, 0]
    return (o[0].transpose(1, 0, 2).reshape(racc, f * ds),
            o[1].transpose(1, 0, 2).reshape(racc, f * ds))


def kernel(x_customer, x_product, x_store, Wc, bc, Wp, bp, Ws, bs, Wl, bl, Wr,
           ln_g, ln_b, edge_index_buys, edge_index_bought_by,
           edge_index_visits, edge_index_visited_by, edge_index_sold_at,
           edge_index_sells):
    f32 = jnp.float32
    npads = (_NCP, _NPP, _NSP)
    nreal = (_NCE, _NPE, _NSE)

    def pad16(x, d, npad):
        o = jnp.zeros((npad, 16), f32)
        o = o.at[:x.shape[0], :d].set(x.astype(f32))
        return o.at[:x.shape[0], d].set(1.0)

    x16 = (pad16(x_customer, 6, _NCP), pad16(x_product, 4, _NPP),
           pad16(x_store, 3, _NSP))

    edges = (edge_index_buys, edge_index_bought_by, edge_index_visits,
             edge_index_visited_by, edge_index_sold_at, edge_index_sells)
    epks, epad = [], []
    for e in range(6):
        ei = edges[e]
        n = ei.shape[1]
        ep = _ceil_to(n, _EUNIT)
        racc = npads[_DST_T[e]]
        s = jnp.concatenate([ei[0].astype(jnp.int32),
                             jnp.zeros((ep - n,), jnp.int32)])
        d = jnp.concatenate([ei[1].astype(jnp.int32),
                             jnp.full((ep - n,), _BIG, jnp.int32)])
        off = jnp.minimum(d, racc)
        s4 = s.reshape(-1, _K, 128)
        o4 = off.reshape(-1, _K, 128)
        epks.append(jnp.concatenate([s4, o4], axis=1).reshape(-1, 128))
        epad.append(ep)

    wpad, bvec = [], (bc, bp, bs)
    for w, d in ((Wc, 6), (Wp, 4), (Ws, 3)):
        wpad.append(jnp.zeros((16, _H), f32).at[:d].set(w.astype(f32)))

    l1p = []
    for e in range(6):
        racc = npads[_DST_T[e]]
        seg = _make_segsum(npads[_SRC_T[e]], 16, 1, epad[e], racc)
        out = seg(x16[_SRC_T[e]], epks[e])
        l1p.append(_unsplit(out, racc, 16, 1))

    xcur = []
    for t in range(3):
        e1, e2 = _INC[t]
        aw, cv, bw, cstv = [], [], [], []
        for e in (e1, e2):
            aw.append(0.5 * (wpad[_SRC_T[e]] @ Wl[0, e]))
            cv.append(0.5 * (bvec[_SRC_T[e]].astype(f32) @ Wl[0, e]))
            bw.append(0.5 * (wpad[t] @ Wr[0, e]))
            cstv.append(0.5 * (bl[0, e] + bvec[t].astype(f32) @ Wr[0, e]))
        stage = _make_stage_a(npads[t], _DCOL[_SRC_T[e1]], _DCOL[_SRC_T[e2]])
        xcur.append(stage(
            x16[t], l1p[e1][0], l1p[e1][1], l1p[e2][0], l1p[e2][1],
            aw[0], aw[1], bw[0] + bw[1], jnp.stack(cv),
            (cstv[0] + cstv[1]).reshape(1, _H),
            ln_g[0, t].reshape(1, _H), ln_b[0, t].reshape(1, _H)))

    l2p = []
    for e in range(6):
        t_dst = _DST_T[e]
        ds, fnum = _L2DS[t_dst]
        racc = npads[t_dst]
        table = _feat_split(xcur[_SRC_T[e]], ds, fnum)
        seg = _make_segsum(npads[_SRC_T[e]], ds, fnum, epad[e], racc)
        l2p.append(seg(table, epks[e]).reshape(2, fnum, racc, ds))

    res = []
    for t in range(3):
        e1, e2 = _INC[t]
        ds, fnum = _L2DS[t]
        wl1 = 0.5 * Wl[1, e1]
        wl2 = 0.5 * Wl[1, e2]
        wrh = 0.5 * (Wr[1, e1] + Wr[1, e2])
        cstv = 0.5 * (bl[1, e1] + bl[1, e2])
        stage = _make_stage_b(npads[t], _DCOL[_SRC_T[e1]], _DCOL[_SRC_T[e2]],
                              ds, fnum)
        slices = [l2p[e][c][ff]
                  for e in (e1, e2) for c in range(2) for ff in range(fnum)]
        h = stage(
            xcur[t], *slices,
            l1p[e1][0], l1p[e1][1], l1p[e2][0], l1p[e2][1],
            wl1, wl2, wrh, cstv.reshape(1, _H),
            ln_g[1, t].reshape(1, _H), ln_b[1, t].reshape(1, _H))
        res.append(h[:nreal[t]])

    return tuple(res)


# speedup vs baseline: 1.1332x; 1.1332x over previous
import functools

import jax
import jax.numpy as jnp
from jax import lax
from jax.experimental import pallas as pl
from jax.experimental.pallas import tpu as pltpu
from jax.experimental.pallas import tpu_sc as plsc

_H = 128
_NCE, _NPE, _NSE = 100000, 50000, 1000
_NCP, _NPP, _NSP = 100096, 50176, 1024
_EUNIT = 32768
_BIG = 1 << 30
_K = 4


def _ceil_to(x, m):
    return -(-x // m) * m


@functools.cache
def _make_segsum(ns_pad, ds, f, ep, racc):
    cr = ep // 4096
    ng = cr // _K
    rps = racc // 16
    zr = max(16, 4096 // ds)
    nzb, nzs = rps // zr, (rps % zr) // 16
    mesh = plsc.VectorSubcoreMesh(core_axis_name="c", subcore_axis_name="s")

    scratch = [
        pltpu.VMEM((2 * _K, 128), jnp.int32),
        pltpu.VMEM((_K, 128, ds), jnp.float32),
        pltpu.VMEM((zr, ds), jnp.float32),
        pltpu.VMEM_SHARED((racc + 16, ds), jnp.float32),
        pltpu.SemaphoreType.DMA,
    ]
    if f > 1:
        scratch.append(pltpu.VMEM((_K, 128), jnp.int32))

    @functools.partial(
        pl.kernel, mesh=mesh,
        out_type=jax.ShapeDtypeStruct((2 * f * racc, ds), jnp.float32),
        scratch_types=scratch,
        compiler_params=pltpu.CompilerParams(use_tc_tiling_on_sc=False),
    )
    def seg(table, epk, out, ebuf, rows, zt, acc, sem_g, *maybe_srcf):
        cid = lax.axis_index("c")
        sid = lax.axis_index("s")
        base_row = cid * (ep // 256) + sid * cr

        for r in range(zr):
            for j in range(ds // 16):
                zt[r, pl.ds(j * 16, 16)] = jnp.zeros((16,), jnp.float32)

        def zero_slice():
            def zb(k, _):
                pltpu.sync_copy(zt, acc.at[pl.ds(sid * rps + k * zr, zr)])
                return 0
            lax.fori_loop(0, nzb, zb, 0)
            for k in range(nzs):
                pltpu.sync_copy(
                    zt.at[pl.ds(0, 16)],
                    acc.at[pl.ds(sid * rps + nzb * zr + k * 16, 16)])

        zero_slice()
        plsc.subcore_barrier()

        def pass_body(p, _):
            foff = p * ns_pad

            def gbody(g, _):
                pltpu.sync_copy(epk.at[pl.ds(2 * base_row + g * 8, 8)], ebuf)
                if f > 1:
                    srcf = maybe_srcf[0]
                    for j in range(_K):
                        for v in range(8):
                            sl = pl.ds(v * 16, 16)
                            srcf[j, sl] = ebuf[j, sl] + foff
                handles = []
                for j in range(_K):
                    idx = maybe_srcf[0].at[j] if f > 1 else ebuf.at[j]
                    handles.append(pltpu.async_copy(
                        table.at[idx], rows.at[j], sem_g))
                for h in handles:
                    h.wait()
                for j in range(_K):
                    pltpu.sync_copy(rows.at[j], acc.at[ebuf.at[_K + j]],
                                    add=True)
                return 0

            lax.fori_loop(0, ng, gbody, 0)
            plsc.subcore_barrier()

            pltpu.sync_copy(
                acc.at[pl.ds(sid * rps, rps)],
                out.at[pl.ds((cid * f + p) * racc + sid * rps, rps)])
            zero_slice()
            plsc.subcore_barrier()
            return 0

        lax.fori_loop(0, f, pass_body, 0)

    return seg


_BLK = 256


def _full(shape):
    return pl.BlockSpec(shape, lambda i: (0, 0))


def _rows(w):
    return pl.BlockSpec((_BLK, w), lambda i: (i, 0))


def _layer_tail(h, g, b):
    mu = jnp.mean(h, axis=1, keepdims=True)
    var = jnp.mean((h - mu) ** 2, axis=1, keepdims=True)
    hn = (h - mu) * lax.rsqrt(var + 1e-5) * g + b
    return jnp.maximum(hn, 0.0)


@functools.cache
def _make_stage_a(npad, dc1, dc2):
    def body(x16, a1c0, a1c1, a2c0, a2c1, aw1, aw2, bwh, cv, cst, g, b, o):
        a1 = a1c0[...] + a1c1[...]
        a2 = a2c0[...] + a2c1[...]
        c1 = a1[:, dc1:dc1 + 1]
        c2 = a2[:, dc2:dc2 + 1]
        na1 = a1 / jnp.maximum(c1, 1.0)
        na2 = a2 / jnp.maximum(c2, 1.0)
        m1 = (c1 > 0).astype(jnp.float32)
        m2 = (c2 > 0).astype(jnp.float32)
        h = (jnp.dot(na1, aw1[...], preferred_element_type=jnp.float32)
             + jnp.dot(na2, aw2[...], preferred_element_type=jnp.float32)
             + jnp.dot(x16[...], bwh[...], preferred_element_type=jnp.float32)
             + m1 * cv[0:1, :] + m2 * cv[1:2, :] + cst[...])
        o[...] = _layer_tail(h, g[...], b[...])

    return pl.pallas_call(
        body,
        grid=(npad // _BLK,),
        in_specs=[_rows(16)] * 5 + [_full((16, _H))] * 3
        + [_full((2, _H)), _full((1, _H)), _full((1, _H)), _full((1, _H))],
        out_specs=_rows(_H),
        out_shape=jax.ShapeDtypeStruct((npad, _H), jnp.float32),
    )


@functools.cache
def _make_stage_b(npad, dc1, dc2):
    def body(x, a1c0, a1c1, a2c0, a2c1, k1c0, k1c1, k2c0, k2c1,
             wl1, wl2, wrh, cst, g, b, o):
        c1 = (k1c0[...] + k1c1[...])[:, dc1:dc1 + 1]
        c2 = (k2c0[...] + k2c1[...])[:, dc2:dc2 + 1]
        na1 = (a1c0[...] + a1c1[...]) / jnp.maximum(c1, 1.0)
        na2 = (a2c0[...] + a2c1[...]) / jnp.maximum(c2, 1.0)
        h = (jnp.dot(na1, wl1[...], preferred_element_type=jnp.float32)
             + jnp.dot(na2, wl2[...], preferred_element_type=jnp.float32)
             + jnp.dot(x[...], wrh[...], preferred_element_type=jnp.float32)
             + cst[...])
        o[...] = _layer_tail(h, g[...], b[...])

    return pl.pallas_call(
        body,
        grid=(npad // _BLK,),
        in_specs=[_rows(_H)] * 5 + [_rows(16)] * 4 + [_full((_H, _H))] * 3
        + [_full((1, _H)), _full((1, _H)), _full((1, _H))],
        out_specs=_rows(_H),
        out_shape=jax.ShapeDtypeStruct((npad, _H), jnp.float32),
    )


_SRC_T = (0, 1, 0, 2, 1, 2)
_DST_T = (1, 0, 2, 0, 2, 1)
_DCOL = (6, 4, 3)
_INC = ((1, 3), (0, 5), (2, 4))
_L2DS = ((16, 8), (32, 4), (128, 1))


def _feat_split(x, ds, f):
    if f == 1:
        return x
    n = x.shape[0]
    return x.reshape(n, f, ds).transpose(1, 0, 2).reshape(f * n, ds)


def _unsplit(out, racc, ds, f):
    o = out.reshape(2, f, racc, ds)
    if f == 1:
        return o[0, 0], o[1, 0]
    return (o[0].transpose(1, 0, 2).reshape(racc, f * ds),
            o[1].transpose(1, 0, 2).reshape(racc, f * ds))


def kernel(x_customer, x_product, x_store, Wc, bc, Wp, bp, Ws, bs, Wl, bl, Wr,
           ln_g, ln_b, edge_index_buys, edge_index_bought_by,
           edge_index_visits, edge_index_visited_by, edge_index_sold_at,
           edge_index_sells):
    f32 = jnp.float32
    npads = (_NCP, _NPP, _NSP)
    nreal = (_NCE, _NPE, _NSE)

    def pad16(x, d, npad):
        o = jnp.zeros((npad, 16), f32)
        o = o.at[:x.shape[0], :d].set(x.astype(f32))
        return o.at[:x.shape[0], d].set(1.0)

    x16 = (pad16(x_customer, 6, _NCP), pad16(x_product, 4, _NPP),
           pad16(x_store, 3, _NSP))

    edges = (edge_index_buys, edge_index_bought_by, edge_index_visits,
             edge_index_visited_by, edge_index_sold_at, edge_index_sells)
    epks, epad = [], []
    for e in range(6):
        ei = edges[e]
        n = ei.shape[1]
        ep = _ceil_to(n, _EUNIT)
        racc = npads[_DST_T[e]]
        s = jnp.concatenate([ei[0].astype(jnp.int32),
                             jnp.zeros((ep - n,), jnp.int32)])
        d = jnp.concatenate([ei[1].astype(jnp.int32),
                             jnp.full((ep - n,), _BIG, jnp.int32)])
        off = jnp.minimum(d, racc)
        s4 = s.reshape(-1, _K, 128)
        o4 = off.reshape(-1, _K, 128)
        epks.append(jnp.concatenate([s4, o4], axis=1).reshape(-1, 128))
        epad.append(ep)

    wpad, bvec = [], (bc, bp, bs)
    for w, d in ((Wc, 6), (Wp, 4), (Ws, 3)):
        wpad.append(jnp.zeros((16, _H), f32).at[:d].set(w.astype(f32)))

    l1p = []
    for e in range(6):
        racc = npads[_DST_T[e]]
        seg = _make_segsum(npads[_SRC_T[e]], 16, 1, epad[e], racc)
        out = seg(x16[_SRC_T[e]], epks[e])
        l1p.append(_unsplit(out, racc, 16, 1))

    xcur = []
    for t in range(3):
        e1, e2 = _INC[t]
        aw, cv, bw, cstv = [], [], [], []
        for e in (e1, e2):
            aw.append(0.5 * (wpad[_SRC_T[e]] @ Wl[0, e]))
            cv.append(0.5 * (bvec[_SRC_T[e]].astype(f32) @ Wl[0, e]))
            bw.append(0.5 * (wpad[t] @ Wr[0, e]))
            cstv.append(0.5 * (bl[0, e] + bvec[t].astype(f32) @ Wr[0, e]))
        stage = _make_stage_a(npads[t], _DCOL[_SRC_T[e1]], _DCOL[_SRC_T[e2]])
        xcur.append(stage(
            x16[t], l1p[e1][0], l1p[e1][1], l1p[e2][0], l1p[e2][1],
            aw[0], aw[1], bw[0] + bw[1], jnp.stack(cv),
            (cstv[0] + cstv[1]).reshape(1, _H),
            ln_g[0, t].reshape(1, _H), ln_b[0, t].reshape(1, _H)))

    l2p = []
    for e in range(6):
        t_dst = _DST_T[e]
        ds, fnum = _L2DS[t_dst]
        racc = npads[t_dst]
        table = _feat_split(xcur[_SRC_T[e]], ds, fnum)
        seg = _make_segsum(npads[_SRC_T[e]], ds, fnum, epad[e], racc)
        out = seg(table, epks[e])
        l2p.append(_unsplit(out, racc, ds, fnum))

    res = []
    for t in range(3):
        e1, e2 = _INC[t]
        wl1 = 0.5 * Wl[1, e1]
        wl2 = 0.5 * Wl[1, e2]
        wrh = 0.5 * (Wr[1, e1] + Wr[1, e2])
        cstv = 0.5 * (bl[1, e1] + bl[1, e2])
        stage = _make_stage_b(npads[t], _DCOL[_SRC_T[e1]], _DCOL[_SRC_T[e2]])
        h = stage(
            xcur[t], l2p[e1][0], l2p[e1][1], l2p[e2][0], l2p[e2][1],
            l1p[e1][0], l1p[e1][1], l1p[e2][0], l1p[e2][1],
            wl1, wl2, wrh, cstv.reshape(1, _H),
            ln_g[1, t].reshape(1, _H), ln_b[1, t].reshape(1, _H))
        res.append(h[:nreal[t]])

    return tuple(res)


# speedup vs baseline: 1.1413x; 1.0072x over previous
import functools

import jax
import jax.numpy as jnp
from jax import lax
from jax.experimental import pallas as pl
from jax.experimental.pallas import tpu as pltpu
from jax.experimental.pallas import tpu_sc as plsc

_H = 128
_NCE, _NPE, _NSE = 100000, 50000, 1000
_NCP, _NPP, _NSP = 100096, 50176, 1024
_EUNIT = 32768
_BIG = 1 << 30
_K = 4


def _ceil_to(x, m):
    return -(-x // m) * m


@functools.cache
def _make_segsum(ns_pad, ds, f, ep, racc, k=_K):
    cr = ep // 4096
    ng = cr // k
    rps = racc // 16
    zr = max(16, 4096 // ds)
    nzb, nzs = rps // zr, (rps % zr) // 16
    mesh = plsc.VectorSubcoreMesh(core_axis_name="c", subcore_axis_name="s")

    scratch = [
        pltpu.VMEM((2 * k, 128), jnp.int32),
        pltpu.VMEM((k, 128, ds), jnp.float32),
        pltpu.VMEM((zr, ds), jnp.float32),
        pltpu.VMEM_SHARED((racc + 16, ds), jnp.float32),
        pltpu.SemaphoreType.DMA,
    ]
    if f > 1:
        scratch.append(pltpu.VMEM((k, 128), jnp.int32))

    @functools.partial(
        pl.kernel, mesh=mesh,
        out_type=jax.ShapeDtypeStruct((2 * f * racc, ds), jnp.float32),
        scratch_types=scratch,
        compiler_params=pltpu.CompilerParams(use_tc_tiling_on_sc=False),
    )
    def seg(table, epk, out, ebuf, rows, zt, acc, sem_g, *maybe_srcf):
        cid = lax.axis_index("c")
        sid = lax.axis_index("s")
        base_row = cid * (ep // 256) + sid * cr

        for r in range(zr):
            for j in range(ds // 16):
                zt[r, pl.ds(j * 16, 16)] = jnp.zeros((16,), jnp.float32)

        def zero_slice():
            def zb(k, _):
                pltpu.sync_copy(zt, acc.at[pl.ds(sid * rps + k * zr, zr)])
                return 0
            lax.fori_loop(0, nzb, zb, 0)
            for k in range(nzs):
                pltpu.sync_copy(
                    zt.at[pl.ds(0, 16)],
                    acc.at[pl.ds(sid * rps + nzb * zr + k * 16, 16)])

        zero_slice()
        plsc.subcore_barrier()

        def pass_body(p, _):
            foff = p * ns_pad

            def gbody(g, _):
                pltpu.sync_copy(
                    epk.at[pl.ds(2 * base_row + g * 2 * k, 2 * k)], ebuf)
                srow = [(j >> 2) * 8 + (j & 3) for j in range(k)]
                if f > 1:
                    srcf = maybe_srcf[0]
                    for j in range(k):
                        for v in range(8):
                            sl = pl.ds(v * 16, 16)
                            srcf[j, sl] = ebuf[srow[j], sl] + foff
                handles = []
                for j in range(k):
                    idx = maybe_srcf[0].at[j] if f > 1 else ebuf.at[srow[j]]
                    handles.append(pltpu.async_copy(
                        table.at[idx], rows.at[j], sem_g))
                for h in handles:
                    h.wait()
                for j in range(k):
                    pltpu.sync_copy(rows.at[j],
                                    acc.at[ebuf.at[srow[j] + 4]], add=True)
                return 0

            lax.fori_loop(0, ng, gbody, 0)
            plsc.subcore_barrier()

            pltpu.sync_copy(
                acc.at[pl.ds(sid * rps, rps)],
                out.at[pl.ds((cid * f + p) * racc + sid * rps, rps)])
            zero_slice()
            plsc.subcore_barrier()
            return 0

        lax.fori_loop(0, f, pass_body, 0)

    return seg


_BLK = 256


def _full(shape):
    return pl.BlockSpec(shape, lambda i: (0, 0))


def _rows(w):
    return pl.BlockSpec((_BLK, w), lambda i: (i, 0))


def _layer_tail(h, g, b):
    mu = jnp.mean(h, axis=1, keepdims=True)
    var = jnp.mean((h - mu) ** 2, axis=1, keepdims=True)
    hn = (h - mu) * lax.rsqrt(var + 1e-5) * g + b
    return jnp.maximum(hn, 0.0)


@functools.cache
def _make_stage_a(npad, dc1, dc2):
    def body(x16, a1c0, a1c1, a2c0, a2c1, aw1, aw2, bwh, cv, cst, g, b, o):
        a1 = a1c0[...] + a1c1[...]
        a2 = a2c0[...] + a2c1[...]
        c1 = a1[:, dc1:dc1 + 1]
        c2 = a2[:, dc2:dc2 + 1]
        na1 = a1 / jnp.maximum(c1, 1.0)
        na2 = a2 / jnp.maximum(c2, 1.0)
        m1 = (c1 > 0).astype(jnp.float32)
        m2 = (c2 > 0).astype(jnp.float32)
        h = (jnp.dot(na1, aw1[...], preferred_element_type=jnp.float32)
             + jnp.dot(na2, aw2[...], preferred_element_type=jnp.float32)
             + jnp.dot(x16[...], bwh[...], preferred_element_type=jnp.float32)
             + m1 * cv[0:1, :] + m2 * cv[1:2, :] + cst[...])
        o[...] = _layer_tail(h, g[...], b[...])

    return pl.pallas_call(
        body,
        grid=(npad // _BLK,),
        in_specs=[_rows(16)] * 5 + [_full((16, _H))] * 3
        + [_full((2, _H)), _full((1, _H)), _full((1, _H)), _full((1, _H))],
        out_specs=_rows(_H),
        out_shape=jax.ShapeDtypeStruct((npad, _H), jnp.float32),
    )


@functools.cache
def _make_stage_b(npad, dc1, dc2):
    def body(x, a1c0, a1c1, a2c0, a2c1, k1c0, k1c1, k2c0, k2c1,
             wl1, wl2, wrh, cst, g, b, o):
        c1 = (k1c0[...] + k1c1[...])[:, dc1:dc1 + 1]
        c2 = (k2c0[...] + k2c1[...])[:, dc2:dc2 + 1]
        na1 = (a1c0[...] + a1c1[...]) / jnp.maximum(c1, 1.0)
        na2 = (a2c0[...] + a2c1[...]) / jnp.maximum(c2, 1.0)
        h = (jnp.dot(na1, wl1[...], preferred_element_type=jnp.float32)
             + jnp.dot(na2, wl2[...], preferred_element_type=jnp.float32)
             + jnp.dot(x[...], wrh[...], preferred_element_type=jnp.float32)
             + cst[...])
        o[...] = _layer_tail(h, g[...], b[...])

    return pl.pallas_call(
        body,
        grid=(npad // _BLK,),
        in_specs=[_rows(_H)] * 5 + [_rows(16)] * 4 + [_full((_H, _H))] * 3
        + [_full((1, _H)), _full((1, _H)), _full((1, _H))],
        out_specs=_rows(_H),
        out_shape=jax.ShapeDtypeStruct((npad, _H), jnp.float32),
    )


_SRC_T = (0, 1, 0, 2, 1, 2)
_DST_T = (1, 0, 2, 0, 2, 1)
_DCOL = (6, 4, 3)
_INC = ((1, 3), (0, 5), (2, 4))
_L2DS = ((16, 8), (32, 4), (128, 1))


def _feat_split(x, ds, f):
    if f == 1:
        return x
    n = x.shape[0]
    return x.reshape(n, f, ds).transpose(1, 0, 2).reshape(f * n, ds)


def _unsplit(out, racc, ds, f):
    o = out.reshape(2, f, racc, ds)
    if f == 1:
        return o[0, 0], o[1, 0]
    return (o[0].transpose(1, 0, 2).reshape(racc, f * ds),
            o[1].transpose(1, 0, 2).reshape(racc, f * ds))


def kernel(x_customer, x_product, x_store, Wc, bc, Wp, bp, Ws, bs, Wl, bl, Wr,
           ln_g, ln_b, edge_index_buys, edge_index_bought_by,
           edge_index_visits, edge_index_visited_by, edge_index_sold_at,
           edge_index_sells):
    f32 = jnp.float32
    npads = (_NCP, _NPP, _NSP)
    nreal = (_NCE, _NPE, _NSE)

    def pad16(x, d, npad):
        o = jnp.zeros((npad, 16), f32)
        o = o.at[:x.shape[0], :d].set(x.astype(f32))
        return o.at[:x.shape[0], d].set(1.0)

    x16 = (pad16(x_customer, 6, _NCP), pad16(x_product, 4, _NPP),
           pad16(x_store, 3, _NSP))

    edges = (edge_index_buys, edge_index_bought_by, edge_index_visits,
             edge_index_visited_by, edge_index_sold_at, edge_index_sells)
    epks, epad = [], []
    for e in range(6):
        ei = edges[e]
        n = ei.shape[1]
        ep = _ceil_to(n, _EUNIT)
        racc = npads[_DST_T[e]]
        s = jnp.concatenate([ei[0].astype(jnp.int32),
                             jnp.zeros((ep - n,), jnp.int32)])
        d = jnp.concatenate([ei[1].astype(jnp.int32),
                             jnp.full((ep - n,), _BIG, jnp.int32)])
        off = jnp.minimum(d, racc)
        s4 = s.reshape(-1, _K, 128)
        o4 = off.reshape(-1, _K, 128)
        epks.append(jnp.concatenate([s4, o4], axis=1).reshape(-1, 128))
        epad.append(ep)

    wpad, bvec = [], (bc, bp, bs)
    for w, d in ((Wc, 6), (Wp, 4), (Ws, 3)):
        wpad.append(jnp.zeros((16, _H), f32).at[:d].set(w.astype(f32)))

    l1p = []
    for e in range(6):
        racc = npads[_DST_T[e]]
        seg = _make_segsum(npads[_SRC_T[e]], 16, 1, epad[e], racc, k=8)
        out = seg(x16[_SRC_T[e]], epks[e])
        l1p.append(_unsplit(out, racc, 16, 1))

    xcur = []
    for t in range(3):
        e1, e2 = _INC[t]
        aw, cv, bw, cstv = [], [], [], []
        for e in (e1, e2):
            aw.append(0.5 * (wpad[_SRC_T[e]] @ Wl[0, e]))
            cv.append(0.5 * (bvec[_SRC_T[e]].astype(f32) @ Wl[0, e]))
            bw.append(0.5 * (wpad[t] @ Wr[0, e]))
            cstv.append(0.5 * (bl[0, e] + bvec[t].astype(f32) @ Wr[0, e]))
        stage = _make_stage_a(npads[t], _DCOL[_SRC_T[e1]], _DCOL[_SRC_T[e2]])
        xcur.append(stage(
            x16[t], l1p[e1][0], l1p[e1][1], l1p[e2][0], l1p[e2][1],
            aw[0], aw[1], bw[0] + bw[1], jnp.stack(cv),
            (cstv[0] + cstv[1]).reshape(1, _H),
            ln_g[0, t].reshape(1, _H), ln_b[0, t].reshape(1, _H)))

    l2p = []
    for e in range(6):
        t_dst = _DST_T[e]
        ds, fnum = _L2DS[t_dst]
        racc = npads[t_dst]
        table = _feat_split(xcur[_SRC_T[e]], ds, fnum)
        seg = _make_segsum(npads[_SRC_T[e]], ds, fnum, epad[e], racc,
                           k=8 if ds == 16 else 4)
        out = seg(table, epks[e])
        l2p.append(_unsplit(out, racc, ds, fnum))

    res = []
    for t in range(3):
        e1, e2 = _INC[t]
        wl1 = 0.5 * Wl[1, e1]
        wl2 = 0.5 * Wl[1, e2]
        wrh = 0.5 * (Wr[1, e1] + Wr[1, e2])
        cstv = 0.5 * (bl[1, e1] + bl[1, e2])
        stage = _make_stage_b(npads[t], _DCOL[_SRC_T[e1]], _DCOL[_SRC_T[e2]])
        h = stage(
            xcur[t], l2p[e1][0], l2p[e1][1], l2p[e2][0], l2p[e2][1],
            l1p[e1][0], l1p[e1][1], l1p[e2][0], l1p[e2][1],
            wl1, wl2, wrh, cstv.reshape(1, _H),
            ln_g[1, t].reshape(1, _H), ln_b[1, t].reshape(1, _H))
        res.append(h[:nreal[t]])

    return tuple(res)
